# baseline (speedup 1.0000x reference)
import functools

import jax
import jax.numpy as jnp
from jax import lax
from jax.experimental import pallas as pl
from jax.experimental.pallas import tpu as pltpu
from jax.experimental.pallas import tpu_sc as plsc

NUM_CORES = 2
NUM_SUBCORES = 16
NUM_WORKERS = NUM_CORES * NUM_SUBCORES
VEC = 16
NBUF = 3
SC_ROWS = 3072
TC_BLOCK = 256


def _sc_add(x, pos_embedding, sp, sc_rows, full_seq_len):
    batch, seq_len, d_model = x.shape
    rows_per_worker = sc_rows // NUM_WORKERS
    chunk = min(8, rows_per_worker)
    n_chunks = rows_per_worker // chunk
    bh = batch // 2
    n_steps = n_chunks * 2
    n_iters = n_steps // 4
    vecs_per_row = d_model // VEC
    n_vecs = 2 * chunk * vecs_per_row
    row_shift = vecs_per_row.bit_length() - 1
    chunk_mask = chunk - 1
    chunk_shift = chunk.bit_length() - 1

    mesh = plsc.VectorSubcoreMesh(
        core_axis_name="c", subcore_axis_name="s",
        num_cores=NUM_CORES, num_subcores=NUM_SUBCORES)

    @functools.partial(
        pl.kernel,
        out_type=jax.ShapeDtypeStruct((batch, full_seq_len, d_model),
                                      jnp.float32),
        mesh=mesh,
        scratch_types=[
            pltpu.VMEM((16,), jnp.int32),
            [pltpu.VMEM((chunk, d_model), jnp.float32)] * 2,
            [pltpu.VMEM((2, chunk, d_model), jnp.float32)] * 4,
            [pltpu.SemaphoreType.DMA] * 2,
            [pltpu.SemaphoreType.DMA] * 4,
            [pltpu.SemaphoreType.DMA] * 4,
        ],
        compiler_params=pltpu.CompilerParams(use_tc_tiling_on_sc=True),
    )
    def run(x_hbm, pos_hbm, sp_hbm, out_hbm, sp_vmem, posbufs, xbufs,
            pos_sems, ld_sems, st_sems):
        cid = lax.axis_index("c")
        sid = lax.axis_index("s")
        wid = sid * NUM_CORES + cid
        pltpu.sync_copy(sp_hbm, sp_vmem)
        s0 = sp_vmem[...][0]
        base = wid * rows_per_worker

        def rows_of(c):
            return pl.multiple_of(base + c * chunk, chunk)

        def issue_pos(c, slot):
            prow = pl.multiple_of(s0 + rows_of(c), 8)
            pltpu.async_copy(pos_hbm.at[pl.ds(prow, chunk)],
                             posbufs[slot], pos_sems[slot])

        def issue_ld(t, slot):
            c = lax.shift_right_logical(t, 1)
            h = lax.bitwise_and(t, 1)
            pltpu.async_copy(
                x_hbm.at[pl.ds(pl.multiple_of(h * 2, 2), 2),
                         pl.ds(rows_of(c), chunk)],
                xbufs[slot], ld_sems[slot])

        def drain_ld(slot):
            pltpu.make_async_copy(
                x_hbm.at[pl.ds(0, 2), pl.ds(0, chunk)],
                xbufs[slot], ld_sems[slot]).wait()

        def drain_st(slot):
            pltpu.make_async_copy(
                xbufs[slot],
                out_hbm.at[pl.ds(0, 2), pl.ds(0, chunk)],
                st_sems[slot]).wait()

        def drain_pos(slot):
            pltpu.make_async_copy(
                pos_hbm.at[pl.ds(0, chunk)], posbufs[slot],
                pos_sems[slot]).wait()

        issue_pos(0, 0)
        issue_pos(1, 1)
        issue_ld(0, 0)
        issue_ld(1, 1)

        def iteration(k, _):
            for j in range(4):
                t = k * 4 + j
                pslot = (j >> 1) & 1
                if j in (0, 2):
                    drain_pos(pslot)
                if j in (0, 1):

                    @pl.when(k >= 1)
                    def _():
                        drain_st((j + 2) % 4)

                    issue_ld(t + 2, (j + 2) % 4)
                else:
                    drain_st((j + 2) % 4)

                    @pl.when(k < n_iters - 1)
                    def _():
                        issue_ld(t + 2, (j + 2) % 4)

                drain_ld(j)
                xbuf, posbuf = xbufs[j], posbufs[pslot]

                @plsc.parallel_loop(0, n_vecs, 1, unroll=8)
                def body(i):
                    b2 = lax.shift_right_logical(i, chunk_shift + 6)
                    r = lax.bitwise_and(lax.shift_right_logical(i, 6),
                                        chunk_mask)
                    col = lax.mul(lax.bitwise_and(i, vecs_per_row - 1),
                                  VEC)
                    xbuf[b2, r, pl.ds(col, VEC)] = (
                        xbuf[b2, r, pl.ds(col, VEC)]
                        + posbuf[r, pl.ds(col, VEC)])

                c = lax.shift_right_logical(t, 1)
                h = lax.bitwise_and(t, 1)
                pltpu.async_copy(
                    xbufs[j],
                    out_hbm.at[pl.ds(pl.multiple_of(h * 2, 2), 2),
                               pl.ds(rows_of(c), chunk)],
                    st_sems[j])
                if j in (1, 3):

                    @pl.when(c + 2 < n_chunks)
                    def _():
                        issue_pos(c + 2, pslot)

            return 0

        lax.fori_loop(0, n_iters, iteration, 0)
        drain_st(2)
        drain_st(3)

    return run(x, pos_embedding, sp)


def _tc_add(x, pos_embedding, sp, row_offset):
    batch, seq_len, d_model = x.shape
    tc_rows = seq_len - row_offset
    n_blocks = tc_rows // TC_BLOCK
    blk0 = row_offset // TC_BLOCK

    def body(sp_ref, x_ref, pos_ref, out_ref):
        out_ref[...] = x_ref[...] + pos_ref[...][None]

    grid_spec = pltpu.PrefetchScalarGridSpec(
        num_scalar_prefetch=1,
        grid=(n_blocks,),
        in_specs=[
            pl.BlockSpec((batch, TC_BLOCK, d_model),
                         lambda i, sp: (0, blk0 + i, 0)),
            pl.BlockSpec(
                (TC_BLOCK, d_model),
                lambda i, sp: (sp[0] // TC_BLOCK + blk0 + i, 0)),
        ],
        out_specs=pl.BlockSpec((batch, TC_BLOCK, d_model),
                               lambda i, sp: (0, i, 0)),
    )
    return pl.pallas_call(
        body,
        grid_spec=grid_spec,
        out_shape=jax.ShapeDtypeStruct((batch, tc_rows, d_model),
                                       jnp.float32),
    )(sp, x, pos_embedding)


def kernel(x, pos_embedding, start_pos):
    batch, seq_len, d_model = x.shape
    sp_vec = jnp.full((16,), start_pos, dtype=jnp.int32)
    sc_rows = min(SC_ROWS, seq_len)
    out = _sc_add(x, pos_embedding, sp_vec, sc_rows, seq_len)
    if sc_rows < seq_len:
        sp_s = jnp.reshape(sp_vec[:1], (1,))
        tc_half = _tc_add(x, pos_embedding, sp_s, sc_rows)
        out = lax.dynamic_update_slice(out, tc_half, (0, sc_rows, 0))
    return out


# speedup vs baseline: 1.0391x; 1.0391x over previous
import functools

import jax
import jax.numpy as jnp
from jax import lax
from jax.experimental import pallas as pl
from jax.experimental.pallas import tpu as pltpu
from jax.experimental.pallas import tpu_sc as plsc

NUM_CORES = 2
NUM_SUBCORES = 16
NUM_WORKERS = NUM_CORES * NUM_SUBCORES
VEC = 16
NBUF = 3
SC_ROWS = 3584
TC_BLOCK = 256


def _sc_add(x, pos_embedding, sp, sc_rows, full_seq_len):
    batch, seq_len, d_model = x.shape
    rows_per_worker = sc_rows // NUM_WORKERS
    chunk = min(8, rows_per_worker)
    n_chunks = rows_per_worker // chunk
    bh = batch // 2
    n_steps = n_chunks * 2
    n_iters = n_steps // 4
    vecs_per_row = d_model // VEC
    n_vecs = 2 * chunk * vecs_per_row
    row_shift = vecs_per_row.bit_length() - 1
    chunk_mask = chunk - 1
    chunk_shift = chunk.bit_length() - 1

    mesh = plsc.VectorSubcoreMesh(
        core_axis_name="c", subcore_axis_name="s",
        num_cores=NUM_CORES, num_subcores=NUM_SUBCORES)

    @functools.partial(
        pl.kernel,
        out_type=jax.ShapeDtypeStruct((batch, full_seq_len, d_model),
                                      jnp.float32),
        mesh=mesh,
        scratch_types=[
            pltpu.VMEM((16,), jnp.int32),
            [pltpu.VMEM((chunk, d_model), jnp.float32)] * 2,
            [pltpu.VMEM((2, chunk, d_model), jnp.float32)] * 4,
            [pltpu.SemaphoreType.DMA] * 2,
            [pltpu.SemaphoreType.DMA] * 4,
            [pltpu.SemaphoreType.DMA] * 4,
        ],
        compiler_params=pltpu.CompilerParams(use_tc_tiling_on_sc=True),
    )
    def run(x_hbm, pos_hbm, sp_hbm, out_hbm, sp_vmem, posbufs, xbufs,
            pos_sems, ld_sems, st_sems):
        cid = lax.axis_index("c")
        sid = lax.axis_index("s")
        wid = sid * NUM_CORES + cid
        pltpu.sync_copy(sp_hbm, sp_vmem)
        s0 = sp_vmem[...][0]
        base = wid * rows_per_worker

        def rows_of(c):
            return pl.multiple_of(base + c * chunk, chunk)

        def issue_pos(c, slot):
            prow = pl.multiple_of(s0 + rows_of(c), 8)
            pltpu.async_copy(pos_hbm.at[pl.ds(prow, chunk)],
                             posbufs[slot], pos_sems[slot])

        def issue_ld(t, slot):
            c = lax.shift_right_logical(t, 1)
            h = lax.bitwise_and(t, 1)
            pltpu.async_copy(
                x_hbm.at[pl.ds(pl.multiple_of(h * 2, 2), 2),
                         pl.ds(rows_of(c), chunk)],
                xbufs[slot], ld_sems[slot])

        def drain_ld(slot):
            pltpu.make_async_copy(
                x_hbm.at[pl.ds(0, 2), pl.ds(0, chunk)],
                xbufs[slot], ld_sems[slot]).wait()

        def drain_st(slot):
            pltpu.make_async_copy(
                xbufs[slot],
                out_hbm.at[pl.ds(0, 2), pl.ds(0, chunk)],
                st_sems[slot]).wait()

        def drain_pos(slot):
            pltpu.make_async_copy(
                pos_hbm.at[pl.ds(0, chunk)], posbufs[slot],
                pos_sems[slot]).wait()

        issue_pos(0, 0)
        issue_pos(1, 1)
        issue_ld(0, 0)
        issue_ld(1, 1)

        def iteration(k, _):
            for j in range(4):
                t = k * 4 + j
                pslot = (j >> 1) & 1
                if j in (0, 2):
                    drain_pos(pslot)
                if j in (0, 1):

                    @pl.when(k >= 1)
                    def _():
                        drain_st((j + 2) % 4)

                    issue_ld(t + 2, (j + 2) % 4)
                else:
                    drain_st((j + 2) % 4)

                    @pl.when(k < n_iters - 1)
                    def _():
                        issue_ld(t + 2, (j + 2) % 4)

                drain_ld(j)
                xbuf, posbuf = xbufs[j], posbufs[pslot]

                @plsc.parallel_loop(0, n_vecs, 1, unroll=8)
                def body(i):
                    b2 = lax.shift_right_logical(i, chunk_shift + 6)
                    r = lax.bitwise_and(lax.shift_right_logical(i, 6),
                                        chunk_mask)
                    col = lax.mul(lax.bitwise_and(i, vecs_per_row - 1),
                                  VEC)
                    xbuf[b2, r, pl.ds(col, VEC)] = (
                        xbuf[b2, r, pl.ds(col, VEC)]
                        + posbuf[r, pl.ds(col, VEC)])

                c = lax.shift_right_logical(t, 1)
                h = lax.bitwise_and(t, 1)
                pltpu.async_copy(
                    xbufs[j],
                    out_hbm.at[pl.ds(pl.multiple_of(h * 2, 2), 2),
                               pl.ds(rows_of(c), chunk)],
                    st_sems[j])
                if j in (1, 3):

                    @pl.when(c + 2 < n_chunks)
                    def _():
                        issue_pos(c + 2, pslot)

            return 0

        lax.fori_loop(0, n_iters, iteration, 0)
        drain_st(2)
        drain_st(3)

    return run(x, pos_embedding, sp)


def _tc_add(x, pos_embedding, sp, row_offset):
    batch, seq_len, d_model = x.shape
    tc_rows = seq_len - row_offset
    n_blocks = tc_rows // TC_BLOCK
    blk0 = row_offset // TC_BLOCK

    def body(sp_ref, x_ref, pos_ref, out_ref):
        out_ref[...] = x_ref[...] + pos_ref[...][None]

    grid_spec = pltpu.PrefetchScalarGridSpec(
        num_scalar_prefetch=1,
        grid=(n_blocks,),
        in_specs=[
            pl.BlockSpec((batch, TC_BLOCK, d_model),
                         lambda i, sp: (0, blk0 + i, 0)),
            pl.BlockSpec(
                (TC_BLOCK, d_model),
                lambda i, sp: (sp[0] // TC_BLOCK + blk0 + i, 0)),
        ],
        out_specs=pl.BlockSpec((batch, TC_BLOCK, d_model),
                               lambda i, sp: (0, i, 0)),
    )
    return pl.pallas_call(
        body,
        grid_spec=grid_spec,
        out_shape=jax.ShapeDtypeStruct((batch, tc_rows, d_model),
                                       jnp.float32),
    )(sp, x, pos_embedding)


def kernel(x, pos_embedding, start_pos):
    batch, seq_len, d_model = x.shape
    sp_vec = jnp.full((16,), start_pos, dtype=jnp.int32)
    sc_rows = min(SC_ROWS, seq_len)
    out = _sc_add(x, pos_embedding, sp_vec, sc_rows, seq_len)
    if sc_rows < seq_len:
        sp_s = jnp.reshape(sp_vec[:1], (1,))
        tc_half = _tc_add(x, pos_embedding, sp_s, sc_rows)
        out = lax.dynamic_update_slice(out, tc_half, (0, sc_rows, 0))
    return out


# speedup vs baseline: 1.1301x; 1.0876x over previous
import functools

import jax
import jax.numpy as jnp
from jax import lax
from jax.experimental import pallas as pl
from jax.experimental.pallas import tpu as pltpu
from jax.experimental.pallas import tpu_sc as plsc

NUM_CORES = 2
NUM_SUBCORES = 16
NUM_WORKERS = NUM_CORES * NUM_SUBCORES
VEC = 16
NBUF = 3
SC_ROWS = 4096
TC_BLOCK = 256


def _sc_add(x, pos_embedding, sp, sc_rows, full_seq_len):
    batch, seq_len, d_model = x.shape
    rows_per_worker = sc_rows // NUM_WORKERS
    chunk = min(8, rows_per_worker)
    n_chunks = rows_per_worker // chunk
    n_steps = n_chunks * batch
    n_slots = 2 * batch
    n_iters = n_steps // n_slots
    vecs_per_row = d_model // VEC
    n_vecs = chunk * vecs_per_row
    chunk_mask = chunk - 1

    mesh = plsc.VectorSubcoreMesh(
        core_axis_name="c", subcore_axis_name="s",
        num_cores=NUM_CORES, num_subcores=NUM_SUBCORES)

    @functools.partial(
        pl.kernel,
        out_type=jax.ShapeDtypeStruct((batch, full_seq_len, d_model),
                                      jnp.float32),
        mesh=mesh,
        scratch_types=[
            pltpu.VMEM((16,), jnp.int32),
            [pltpu.VMEM((chunk, d_model), jnp.float32)] * 2,
            [pltpu.VMEM((chunk, d_model), jnp.float32)] * n_slots,
            [pltpu.SemaphoreType.DMA] * 2,
            [pltpu.SemaphoreType.DMA] * n_slots,
            [pltpu.SemaphoreType.DMA] * n_slots,
        ],
        compiler_params=pltpu.CompilerParams(use_tc_tiling_on_sc=True),
    )
    def run(x_hbm, pos_hbm, sp_hbm, out_hbm, sp_vmem, posbufs, xbufs,
            pos_sems, ld_sems, st_sems):
        cid = lax.axis_index("c")
        sid = lax.axis_index("s")
        wid = sid * NUM_CORES + cid
        pltpu.sync_copy(sp_hbm, sp_vmem)
        s0 = sp_vmem[...][0]
        base = wid * rows_per_worker

        def rows_of(c):
            return pl.multiple_of(base + c * chunk, chunk)

        def issue_pos(c, slot):
            prow = pl.multiple_of(s0 + rows_of(c), 8)
            pltpu.async_copy(pos_hbm.at[pl.ds(prow, chunk)],
                             posbufs[slot], pos_sems[slot])

        def issue_ld(t, b, slot):
            c = lax.shift_right_logical(t, 2)
            pltpu.async_copy(
                x_hbm.at[b, pl.ds(rows_of(c), chunk)],
                xbufs[slot], ld_sems[slot])

        def drain_ld(slot):
            pltpu.make_async_copy(
                x_hbm.at[0, pl.ds(0, chunk)],
                xbufs[slot], ld_sems[slot]).wait()

        def drain_st(slot):
            pltpu.make_async_copy(
                xbufs[slot], out_hbm.at[0, pl.ds(0, chunk)],
                st_sems[slot]).wait()

        def drain_pos(slot):
            pltpu.make_async_copy(
                pos_hbm.at[pl.ds(0, chunk)], posbufs[slot],
                pos_sems[slot]).wait()

        issue_pos(0, 0)
        issue_pos(1, 1)
        for b in range(batch):
            issue_ld(b, b, b)

        half = n_slots // 2

        def iteration(k, _):
            for j in range(n_slots):
                t = k * n_slots + j
                b = j % batch
                pslot = (j >= half) * 1
                if j % half == 0:
                    drain_pos(pslot)
                refill = (j + half) % n_slots
                if j < half:

                    @pl.when(k >= 1)
                    def _():
                        drain_st(refill)

                    issue_ld(t + half, b, refill)
                else:
                    drain_st(refill)

                    @pl.when(k < n_iters - 1)
                    def _():
                        issue_ld(t + half, b, refill)

                drain_ld(j)
                xbuf, posbuf = xbufs[j], posbufs[pslot]

                @plsc.parallel_loop(0, n_vecs, 1, unroll=8)
                def body(i):
                    r = lax.bitwise_and(lax.shift_right_logical(i, 6),
                                        chunk_mask)
                    col = lax.mul(lax.bitwise_and(i, vecs_per_row - 1),
                                  VEC)
                    xbuf[r, pl.ds(col, VEC)] = (
                        xbuf[r, pl.ds(col, VEC)]
                        + posbuf[r, pl.ds(col, VEC)])

                c = lax.shift_right_logical(t, 2)
                pltpu.async_copy(
                    xbufs[j], out_hbm.at[b, pl.ds(rows_of(c), chunk)],
                    st_sems[j])
                if j % half == half - 1:

                    @pl.when(lax.shift_right_logical(t, 2) + 2 < n_chunks)
                    def _():
                        issue_pos(lax.shift_right_logical(t, 2) + 2,
                                  pslot)

            return 0

        lax.fori_loop(0, n_iters, iteration, 0)
        for s in range(half, n_slots):
            drain_st(s)

    return run(x, pos_embedding, sp)


def _tc_add(x, pos_embedding, sp, row_offset):
    batch, seq_len, d_model = x.shape
    tc_rows = seq_len - row_offset
    n_blocks = tc_rows // TC_BLOCK
    blk0 = row_offset // TC_BLOCK

    def body(sp_ref, x_ref, pos_ref, out_ref):
        out_ref[...] = x_ref[...] + pos_ref[...][None]

    grid_spec = pltpu.PrefetchScalarGridSpec(
        num_scalar_prefetch=1,
        grid=(n_blocks,),
        in_specs=[
            pl.BlockSpec((batch, TC_BLOCK, d_model),
                         lambda i, sp: (0, blk0 + i, 0)),
            pl.BlockSpec(
                (TC_BLOCK, d_model),
                lambda i, sp: (sp[0] // TC_BLOCK + blk0 + i, 0)),
        ],
        out_specs=pl.BlockSpec((batch, TC_BLOCK, d_model),
                               lambda i, sp: (0, i, 0)),
    )
    return pl.pallas_call(
        body,
        grid_spec=grid_spec,
        out_shape=jax.ShapeDtypeStruct((batch, tc_rows, d_model),
                                       jnp.float32),
    )(sp, x, pos_embedding)


def kernel(x, pos_embedding, start_pos):
    batch, seq_len, d_model = x.shape
    sp_vec = jnp.full((16,), start_pos, dtype=jnp.int32)
    sc_rows = min(SC_ROWS, seq_len)
    out = _sc_add(x, pos_embedding, sp_vec, sc_rows, seq_len)
    if sc_rows < seq_len:
        sp_s = jnp.reshape(sp_vec[:1], (1,))
        tc_half = _tc_add(x, pos_embedding, sp_s, sc_rows)
        out = lax.dynamic_update_slice(out, tc_half, (0, sc_rows, 0))
    return out
